# baseline (speedup 1.0000x reference)
import functools

import jax
import jax.numpy as jnp
from jax import lax
from jax.experimental import pallas as pl
from jax.experimental.pallas import tpu as pltpu
from jax.experimental.pallas import tpu_sc as plsc

VOCAB = 1000000
EMB = 64
BATCH = 4096
SEQ = 200

NW = 32
BPW = BATCH // NW
CHUNK = 40
NCHUNK = BPW * SEQ // CHUNK
EPB = 2
CPE = SEQ // CHUNK
KB = EPB * CPE
NSUPER = BPW // EPB
NPAIR = NSUPER // 2

_mesh = plsc.VectorSubcoreMesh(core_axis_name="c", subcore_axis_name="s")


@functools.partial(
    pl.kernel,
    out_type=jax.ShapeDtypeStruct((BATCH, SEQ, EMB), jnp.float32),
    mesh=_mesh,
    scratch_types=[
        pltpu.VMEM((NCHUNK, CHUNK), jnp.int32),
        pltpu.VMEM((EPB, SEQ, EMB), jnp.float32),
        pltpu.VMEM((EPB, SEQ, EMB), jnp.float32),
        pltpu.SemaphoreType.DMA,
        pltpu.SemaphoreType.DMA,
        pltpu.SemaphoreType.DMA,
        pltpu.SemaphoreType.DMA,
    ],
    compiler_params=pltpu.CompilerParams(use_tc_tiling_on_sc=False),
)
def _embed_gather(idx_hbm, table_hbm, out_hbm, idx_v, buf0, buf1,
                  gsem0, gsem1, ssem0, ssem1):
    cid = lax.axis_index("c")
    sid = lax.axis_index("s")
    wid = sid * 2 + cid
    ebase = wid * BPW

    pltpu.sync_copy(idx_hbm.at[wid], idx_v)

    bufs = (buf0, buf1)
    gsems = (gsem0, gsem1)
    ssems = (ssem0, ssem1)

    def body(t, carry):
        for b in range(2):
            s = 2 * t + b
            buf, gsem, ssem = bufs[b], gsems[b], ssems[b]

            @pl.when(t > 0)
            def _():
                pltpu.make_async_copy(
                    buf, out_hbm.at[pl.ds(ebase + s * EPB, EPB)], ssem
                ).wait()

            handles = [
                pltpu.async_copy(
                    table_hbm.at[idx_v.at[s * KB + c]],
                    buf.at[c // CPE, pl.ds((c % CPE) * CHUNK, CHUNK)],
                    gsem,
                )
                for c in range(KB)
            ]
            for h in handles:
                h.wait()

            pltpu.async_copy(buf, out_hbm.at[pl.ds(ebase + s * EPB, EPB)], ssem)
        return carry

    lax.fori_loop(0, NPAIR, body, 0)

    for b in range(2):
        pltpu.make_async_copy(
            bufs[b], out_hbm.at[pl.ds(ebase, EPB)], ssems[b]
        ).wait()


def kernel(inp, hidden, embed_weight):
    del hidden
    idx = inp.astype(jnp.int32).reshape(NW, NCHUNK, CHUNK)
    return _embed_gather(idx, embed_weight)
